# baseline (speedup 1.0000x reference)
import jax
import jax.numpy as jnp
from jax import lax
from jax.experimental import pallas as pl
from jax.experimental.pallas import tpu as pltpu
from jax.experimental.pallas import tpu_sc as plsc

N_NODES = 10000
N_EDGES = 320000
D = 128

NC = 2
NS = 16
DH = D // NC
CH = 128
NCH = 160
EPT = CH * NCH
E_PAD = EPT * NS
ROWS_PER_TILE = 632
N_PAD = ROWS_PER_TILE * NS

NBUF = 4
LEAD = 2
NH = 4
NCH_H = NCH // NH


def _agg_body(x_hbm, src_hbm, dst_hbm, zeros_hbm, out_hbm,
              src_v, dst_v, rows_v, x_sp, acc, *sems):
    cid = lax.axis_index("c")
    sid = lax.axis_index("s")
    slab = pl.ds(sid * ROWS_PER_TILE, ROWS_PER_TILE)
    cols = pl.ds(cid * DH, DH)

    pltpu.sync_copy(zeros_hbm.at[:, pl.ds(0, DH)], acc.at[slab])
    pltpu.sync_copy(x_hbm.at[slab, cols], x_sp.at[slab])
    plsc.subcore_barrier()

    gsems = sems[:NBUF]
    ssems = sems[NBUF:]

    def wait_gather(k, b):
        pltpu.make_async_copy(
            x_sp.at[src_v.at[k]], rows_v.at[b], gsems[b]).wait()

    def wait_scatter(b):
        pltpu.make_async_copy(
            rows_v.at[b], acc.at[dst_v.at[0]], ssems[b]).wait()

    for h in range(NH):
        pltpu.sync_copy(src_hbm.at[sid, pl.ds(h * NCH_H, NCH_H)], src_v)
        pltpu.sync_copy(dst_hbm.at[sid, pl.ds(h * NCH_H, NCH_H)], dst_v)
        for b in range(LEAD):
            pltpu.async_copy(x_sp.at[src_v.at[b]], rows_v.at[b], gsems[b])

        def ring(j, carry):
            for u in range(NBUF):
                k = j * NBUF + u
                b = u
                wait_gather(k, b)
                pltpu.async_copy(
                    rows_v.at[b], acc.at[dst_v.at[k]], ssems[b], add=True)
                nb = (u + LEAD) % NBUF

                @pl.when(k + LEAD < NCH_H)
                def _():
                    @pl.when(k + LEAD >= NBUF)
                    def _():
                        wait_scatter(nb)
                    pltpu.async_copy(
                        x_sp.at[src_v.at[k + LEAD]], rows_v.at[nb], gsems[nb])
            return carry

        lax.fori_loop(0, NCH_H // NBUF, ring, 0)
        for k in range(NCH_H - NBUF, NCH_H):
            wait_scatter(k % NBUF)
    plsc.subcore_barrier()

    pltpu.sync_copy(acc.at[slab], out_hbm.at[slab, cols])


_agg = pl.kernel(
    _agg_body,
    out_type=jax.ShapeDtypeStruct((N_PAD, D), jnp.float32),
    mesh=plsc.VectorSubcoreMesh(core_axis_name="c", subcore_axis_name="s"),
    scratch_types=[
        pltpu.VMEM((NCH_H, CH), jnp.int32),
        pltpu.VMEM((NCH_H, CH), jnp.int32),
        pltpu.VMEM((NBUF, CH, DH), jnp.float32),
        pltpu.VMEM_SHARED((N_PAD, DH), jnp.float32),
        pltpu.VMEM_SHARED((N_PAD, DH), jnp.float32),
    ] + [pltpu.SemaphoreType.DMA] * (2 * NBUF),
    compiler_params=pltpu.CompilerParams(use_tc_tiling_on_sc=False),
)


def _mm_body(p_ref, w_ref, o_ref):
    o_ref[...] = jnp.dot(p_ref[...], w_ref[...],
                         preferred_element_type=jnp.float32)


_BM = 1000


def _combine_matmul(agg, W):
    return pl.pallas_call(
        _mm_body,
        grid=(N_NODES // _BM,),
        in_specs=[
            pl.BlockSpec((_BM, D), lambda i: (i, 0)),
            pl.BlockSpec((D, D), lambda i: (0, 0)),
        ],
        out_specs=pl.BlockSpec((_BM, D), lambda i: (i, 0)),
        out_shape=jax.ShapeDtypeStruct((N_NODES, D), jnp.float32),
    )(agg, W)


@jax.jit
def kernel(x, edge_index, W):
    src = edge_index[0].astype(jnp.int32)
    dst = edge_index[1].astype(jnp.int32)
    pad = E_PAD - N_EDGES
    src_p = jnp.concatenate([src, jnp.zeros((pad,), jnp.int32)])
    dst_p = jnp.concatenate([dst, jnp.full((pad,), N_NODES, jnp.int32)])
    src_p = src_p.reshape(NS, NCH, CH)
    dst_p = dst_p.reshape(NS, NCH, CH)
    x_pad = jnp.pad(x, ((0, N_PAD - N_NODES), (0, 0)))
    zeros = jnp.zeros((ROWS_PER_TILE, D), jnp.float32)
    agg = _agg(x_pad, src_p, dst_p, zeros)
    return _combine_matmul(agg, W)


# speedup vs baseline: 1.0023x; 1.0023x over previous
import jax
import jax.numpy as jnp
from jax import lax
from jax.experimental import pallas as pl
from jax.experimental.pallas import tpu as pltpu
from jax.experimental.pallas import tpu_sc as plsc

N_NODES = 10000
N_EDGES = 320000
D = 128

NC = 2
NS = 16
DH = D // NC
CH = 64
NCH = 320
EPT = CH * NCH
E_PAD = EPT * NS
ROWS_PER_TILE = 632
N_PAD = ROWS_PER_TILE * NS

NBUF = 8
LEAD = 4
NH = 4
NCH_H = NCH // NH


def _agg_body(x_hbm, src_hbm, dst_hbm, zeros_hbm, out_hbm,
              src_v, dst_v, rows_v, x_sp, acc, *sems):
    cid = lax.axis_index("c")
    sid = lax.axis_index("s")
    slab = pl.ds(sid * ROWS_PER_TILE, ROWS_PER_TILE)
    cols = pl.ds(cid * DH, DH)

    pltpu.sync_copy(zeros_hbm.at[:, pl.ds(0, DH)], acc.at[slab])
    pltpu.sync_copy(x_hbm.at[slab, cols], x_sp.at[slab])
    plsc.subcore_barrier()

    gsems = sems[:NBUF]
    ssems = sems[NBUF:]

    def wait_gather(k, b):
        pltpu.make_async_copy(
            x_sp.at[src_v.at[k]], rows_v.at[b], gsems[b]).wait()

    def wait_scatter(b):
        pltpu.make_async_copy(
            rows_v.at[b], acc.at[dst_v.at[0]], ssems[b]).wait()

    for h in range(NH):
        pltpu.sync_copy(src_hbm.at[sid, pl.ds(h * NCH_H, NCH_H)], src_v)
        pltpu.sync_copy(dst_hbm.at[sid, pl.ds(h * NCH_H, NCH_H)], dst_v)
        for b in range(LEAD):
            pltpu.async_copy(x_sp.at[src_v.at[b]], rows_v.at[b], gsems[b])

        def ring(j, carry):
            for u in range(NBUF):
                k = j * NBUF + u
                b = u
                wait_gather(k, b)
                pltpu.async_copy(
                    rows_v.at[b], acc.at[dst_v.at[k]], ssems[b], add=True)
                nb = (u + LEAD) % NBUF

                @pl.when(k + LEAD < NCH_H)
                def _():
                    @pl.when(k + LEAD >= NBUF)
                    def _():
                        wait_scatter(nb)
                    pltpu.async_copy(
                        x_sp.at[src_v.at[k + LEAD]], rows_v.at[nb], gsems[nb])
            return carry

        lax.fori_loop(0, NCH_H // NBUF, ring, 0)
        for k in range(NCH_H - NBUF, NCH_H):
            wait_scatter(k % NBUF)
    plsc.subcore_barrier()

    pltpu.sync_copy(acc.at[slab], out_hbm.at[slab, cols])


_agg = pl.kernel(
    _agg_body,
    out_type=jax.ShapeDtypeStruct((N_PAD, D), jnp.float32),
    mesh=plsc.VectorSubcoreMesh(core_axis_name="c", subcore_axis_name="s"),
    scratch_types=[
        pltpu.VMEM((NCH_H, CH), jnp.int32),
        pltpu.VMEM((NCH_H, CH), jnp.int32),
        pltpu.VMEM((NBUF, CH, DH), jnp.float32),
        pltpu.VMEM_SHARED((N_PAD, DH), jnp.float32),
        pltpu.VMEM_SHARED((N_PAD, DH), jnp.float32),
    ] + [pltpu.SemaphoreType.DMA] * (2 * NBUF),
    compiler_params=pltpu.CompilerParams(use_tc_tiling_on_sc=False),
)


def _mm_body(p_ref, w_ref, o_ref):
    o_ref[...] = jnp.dot(p_ref[...], w_ref[...],
                         preferred_element_type=jnp.float32)


_BM = 1000


def _combine_matmul(agg, W):
    return pl.pallas_call(
        _mm_body,
        grid=(N_NODES // _BM,),
        in_specs=[
            pl.BlockSpec((_BM, D), lambda i: (i, 0)),
            pl.BlockSpec((D, D), lambda i: (0, 0)),
        ],
        out_specs=pl.BlockSpec((_BM, D), lambda i: (i, 0)),
        out_shape=jax.ShapeDtypeStruct((N_NODES, D), jnp.float32),
    )(agg, W)


@jax.jit
def kernel(x, edge_index, W):
    src = edge_index[0].astype(jnp.int32)
    dst = edge_index[1].astype(jnp.int32)
    pad = E_PAD - N_EDGES
    src_p = jnp.concatenate([src, jnp.zeros((pad,), jnp.int32)])
    dst_p = jnp.concatenate([dst, jnp.full((pad,), N_NODES, jnp.int32)])
    src_p = src_p.reshape(NS, NCH, CH)
    dst_p = dst_p.reshape(NS, NCH, CH)
    x_pad = jnp.pad(x, ((0, N_PAD - N_NODES), (0, 0)))
    zeros = jnp.zeros((ROWS_PER_TILE, D), jnp.float32)
    agg = _agg(x_pad, src_p, dst_p, zeros)
    return _combine_matmul(agg, W)


# speedup vs baseline: 1.0492x; 1.0469x over previous
import jax
import jax.numpy as jnp
from jax import lax
from jax.experimental import pallas as pl
from jax.experimental.pallas import tpu as pltpu
from jax.experimental.pallas import tpu_sc as plsc

N_NODES = 10000
N_EDGES = 320000
D = 128

NC = 2
NS = 16
DH = D // NC
CH = 64
NCH = 320
EPT = CH * NCH
E_PAD = EPT * NS
ROWS_PER_TILE = 632
N_PAD = ROWS_PER_TILE * NS

NBUF = 8
LEAD = 4
NH = 4
NCH_H = NCH // NH


XROWS = N_NODES // NS


def _agg_body(x_hbm, src_hbm, dst_hbm, out_hbm,
              src_v, dst_v, rows_v, x_sp, acc, *sems):
    cid = lax.axis_index("c")
    sid = lax.axis_index("s")
    slab = pl.ds(sid * ROWS_PER_TILE, ROWS_PER_TILE)
    cols = pl.ds(cid * DH, DH)
    xslab = pl.ds(sid * XROWS, XROWS)

    cpx = pltpu.async_copy(x_hbm.at[xslab, cols], x_sp.at[xslab], sems[0])

    zrow = jnp.zeros((16,), jnp.float32)

    def zero_row(r, carry):
        for c4 in range(DH // 16):
            rows_v[0, r, pl.ds(c4 * 16, 16)] = zrow
        return carry

    lax.fori_loop(0, CH, zero_row, 0)
    for i in range(ROWS_PER_TILE // CH):
        pltpu.sync_copy(rows_v.at[0],
                        acc.at[pl.ds(sid * ROWS_PER_TILE + i * CH, CH)])
    _rem = ROWS_PER_TILE % CH
    if _rem:
        pltpu.sync_copy(
            rows_v.at[0, pl.ds(0, _rem)],
            acc.at[pl.ds(sid * ROWS_PER_TILE + ROWS_PER_TILE - _rem, _rem)])
    cpx.wait()
    plsc.subcore_barrier()

    gsems = sems[:NBUF]
    ssems = sems[NBUF:]

    def wait_gather(k, b):
        pltpu.make_async_copy(
            x_sp.at[src_v.at[k]], rows_v.at[b], gsems[b]).wait()

    def wait_scatter(b):
        pltpu.make_async_copy(
            rows_v.at[b], acc.at[dst_v.at[0]], ssems[b]).wait()

    for h in range(NH):
        pltpu.sync_copy(src_hbm.at[sid, pl.ds(h * NCH_H, NCH_H)], src_v)
        pltpu.sync_copy(dst_hbm.at[sid, pl.ds(h * NCH_H, NCH_H)], dst_v)
        for b in range(LEAD):
            pltpu.async_copy(x_sp.at[src_v.at[b]], rows_v.at[b], gsems[b])

        def ring(j, carry):
            for u in range(NBUF):
                k = j * NBUF + u
                b = u
                wait_gather(k, b)
                pltpu.async_copy(
                    rows_v.at[b], acc.at[dst_v.at[k]], ssems[b], add=True)
                nb = (u + LEAD) % NBUF

                @pl.when(k + LEAD < NCH_H)
                def _():
                    @pl.when(k + LEAD >= NBUF)
                    def _():
                        wait_scatter(nb)
                    pltpu.async_copy(
                        x_sp.at[src_v.at[k + LEAD]], rows_v.at[nb], gsems[nb])
            return carry

        lax.fori_loop(0, NCH_H // NBUF, ring, 0)
        for k in range(NCH_H - NBUF, NCH_H):
            wait_scatter(k % NBUF)
    plsc.subcore_barrier()

    pltpu.sync_copy(acc.at[slab], out_hbm.at[slab, cols])


_agg = pl.kernel(
    _agg_body,
    out_type=jax.ShapeDtypeStruct((N_PAD, D), jnp.float32),
    mesh=plsc.VectorSubcoreMesh(core_axis_name="c", subcore_axis_name="s"),
    scratch_types=[
        pltpu.VMEM((NCH_H, CH), jnp.int32),
        pltpu.VMEM((NCH_H, CH), jnp.int32),
        pltpu.VMEM((NBUF, CH, DH), jnp.float32),
        pltpu.VMEM_SHARED((N_PAD, DH), jnp.float32),
        pltpu.VMEM_SHARED((N_PAD, DH), jnp.float32),
    ] + [pltpu.SemaphoreType.DMA] * (2 * NBUF),
    compiler_params=pltpu.CompilerParams(use_tc_tiling_on_sc=False),
)


def _mm_body(p_ref, w_ref, o_ref):
    o_ref[...] = jnp.dot(p_ref[...], w_ref[...],
                         preferred_element_type=jnp.float32)


_BM = 1000


def _combine_matmul(agg, W):
    return pl.pallas_call(
        _mm_body,
        grid=(N_NODES // _BM,),
        in_specs=[
            pl.BlockSpec((_BM, D), lambda i: (i, 0)),
            pl.BlockSpec((D, D), lambda i: (0, 0)),
        ],
        out_specs=pl.BlockSpec((_BM, D), lambda i: (i, 0)),
        out_shape=jax.ShapeDtypeStruct((N_NODES, D), jnp.float32),
    )(agg, W)


@jax.jit
def kernel(x, edge_index, W):
    src = edge_index[0].astype(jnp.int32)
    dst = edge_index[1].astype(jnp.int32)
    pad = E_PAD - N_EDGES
    src_p = jnp.concatenate([src, jnp.zeros((pad,), jnp.int32)])
    dst_p = jnp.concatenate([dst, jnp.full((pad,), N_NODES, jnp.int32)])
    src_p = src_p.reshape(NS, NCH, CH)
    dst_p = dst_p.reshape(NS, NCH, CH)
    agg = _agg(x, src_p, dst_p)
    return _combine_matmul(agg, W)
